# baseline (speedup 1.0000x reference)
import functools
import jax
import jax.numpy as jnp
from jax import lax
from jax.experimental import pallas as pl
from jax.experimental.pallas import tpu as pltpu
from jax.experimental.pallas import tpu_sc as plsc

NUM_WORKERS = 32
NBUF = 4
IDX_SPLIT = 2

TC_BATCHES = 640
TC_BLOCK = 128


def _make_sc_kernel(nb, l, d):
    bpw = nb // NUM_WORKERS
    half = l // IDX_SPLIT
    mesh = plsc.VectorSubcoreMesh(core_axis_name="c", subcore_axis_name="s")

    @functools.partial(
        pl.kernel,
        mesh=mesh,
        out_type=jax.ShapeDtypeStruct((nb * l, d), jnp.float32),
        scratch_types=[
            pltpu.VMEM((bpw * IDX_SPLIT, half), jnp.int32),
            pltpu.VMEM_SHARED((l, d), jnp.float32),
        ]
        + [pltpu.VMEM((l, d), jnp.float32) for _ in range(NBUF)]
        + [pltpu.SemaphoreType.DMA] * (3 * NBUF),
    )
    def sc_kernel(x_hbm, pos_hbm, idx_hbm, out_hbm, idx_v, pos_sh, *rest):
        bufs = rest[:NBUF]
        s_init = rest[NBUF:2 * NBUF]
        s_gadd = rest[2 * NBUF:3 * NBUF]
        s_out = rest[3 * NBUF:4 * NBUF]
        cid = lax.axis_index("c")
        sid = lax.axis_index("s")
        wid = sid * 2 + cid

        @pl.when(sid == 0)
        def _():
            pltpu.sync_copy(pos_hbm, pos_sh)

        pltpu.sync_copy(
            idx_hbm.at[pl.ds(wid * bpw * IDX_SPLIT, bpw * IDX_SPLIT)], idx_v)
        plsc.subcore_barrier()

        init_h = [None] * bpw
        gadd_h = [None] * bpw
        scat_h = [None] * bpw

        def issue_init(j):
            init_h[j] = pltpu.async_copy(pos_sh, bufs[j % NBUF],
                                         s_init[j % NBUF])

        def issue_scat(j):
            scat_h[j] = pltpu.async_copy(
                bufs[j % NBUF],
                out_hbm.at[pl.ds((wid * bpw + j) * l, l)],
                s_out[j % NBUF])

        issue_init(0)
        for i in range(bpw):
            p = i % NBUF
            if i + 1 < bpw:
                if i >= NBUF - 1:
                    scat_h[i - (NBUF - 1)].wait()
                issue_init(i + 1)
            if i >= 1:
                gadd_h[i - 1][0].wait()
                gadd_h[i - 1][1].wait()
                issue_scat(i - 1)
            init_h[i].wait()
            gadd_h[i] = (
                pltpu.async_copy(
                    x_hbm.at[idx_v.at[IDX_SPLIT * i]],
                    bufs[p].at[pl.ds(0, half)], s_gadd[p], add=True),
                pltpu.async_copy(
                    x_hbm.at[idx_v.at[IDX_SPLIT * i + 1]],
                    bufs[p].at[pl.ds(half, half)], s_gadd[p], add=True),
            )
        gadd_h[bpw - 1][0].wait()
        gadd_h[bpw - 1][1].wait()
        issue_scat(bpw - 1)
        for j in range(max(bpw - NBUF + 1, 0), bpw):
            scat_h[j].wait()

    return sc_kernel


def _tc_add_kernel(x_ref, pos_ref, out_ref):
    out_ref[...] = x_ref[...] + pos_ref[...]


def kernel(x, pos_table):
    b, l, d = x.shape
    nb_sc = b - TC_BATCHES

    x2 = x.reshape(b * l, d)
    idx = jnp.arange(TC_BATCHES * l, b * l, dtype=jnp.int32).reshape(
        nb_sc * IDX_SPLIT, l // IDX_SPLIT)
    sc_out = _make_sc_kernel(nb_sc, l, d)(x2, pos_table, idx)

    tc_full = pl.pallas_call(
        _tc_add_kernel,
        grid=(TC_BATCHES // TC_BLOCK,),
        in_specs=[
            pl.BlockSpec((TC_BLOCK, l, d), lambda i: (i, 0, 0)),
            pl.BlockSpec((l, d), lambda i: (0, 0)),
        ],
        out_specs=pl.BlockSpec((TC_BLOCK, l, d), lambda i: (i, 0, 0)),
        out_shape=jax.ShapeDtypeStruct((b, l, d), x.dtype),
    )(x, pos_table)

    return lax.dynamic_update_slice(
        tc_full, sc_out.reshape(nb_sc, l, d), (TC_BATCHES, 0, 0))


# speedup vs baseline: 1.0765x; 1.0765x over previous
import functools
import jax
import jax.numpy as jnp
from jax import lax
from jax.experimental import pallas as pl
from jax.experimental.pallas import tpu as pltpu
from jax.experimental.pallas import tpu_sc as plsc

NUM_WORKERS = 32
NBUF = 2
WIDE = 2
PAIR = 2


def _make_sc_kernel(b, lw, d):
    bpw = b // NUM_WORKERS
    ppw = bpw // PAIR
    mesh = plsc.VectorSubcoreMesh(core_axis_name="c", subcore_axis_name="s")

    @functools.partial(
        pl.kernel,
        mesh=mesh,
        out_type=jax.ShapeDtypeStruct((b * lw, WIDE, d), jnp.float32),
        scratch_types=[
            pltpu.VMEM((ppw, 104), jnp.int32),
            pltpu.VMEM((ppw, 96), jnp.int32),
            pltpu.VMEM_SHARED((PAIR * lw, WIDE, d), jnp.float32),
        ]
        + [pltpu.VMEM((PAIR * lw, WIDE, d), jnp.float32) for _ in range(NBUF)]
        + [pltpu.SemaphoreType.DMA] * (3 * NBUF),
    )
    def sc_kernel(x_hbm, pos_hbm, idx_a_hbm, idx_b_hbm, out_hbm, idx_a_v,
                  idx_b_v, pos_sh, *rest):
        bufs = rest[:NBUF]
        s_init = rest[NBUF:2 * NBUF]
        s_gadd = rest[2 * NBUF:3 * NBUF]
        s_out = rest[3 * NBUF:4 * NBUF]
        cid = lax.axis_index("c")
        sid = lax.axis_index("s")
        wid = sid * 2 + cid

        @pl.when(sid == 0)
        def _():
            pltpu.sync_copy(pos_hbm, pos_sh)

        pltpu.sync_copy(idx_a_hbm.at[pl.ds(wid * ppw, ppw)], idx_a_v)
        pltpu.sync_copy(idx_b_hbm.at[pl.ds(wid * ppw, ppw)], idx_b_v)
        plsc.subcore_barrier()

        init_h = [None] * ppw
        gadd_h = [None] * ppw
        scat_h = [None] * ppw

        def issue_init(j):
            init_h[j] = pltpu.async_copy(pos_sh, bufs[j % NBUF],
                                         s_init[j % NBUF])

        def issue_scat(j):
            scat_h[j] = pltpu.async_copy(
                bufs[j % NBUF],
                out_hbm.at[pl.ds((wid * bpw + j * PAIR) * lw, PAIR * lw)],
                s_out[j % NBUF])

        issue_init(0)
        for i in range(ppw):
            p = i % NBUF
            if i >= 1:
                gadd_h[i - 1][0].wait()
                gadd_h[i - 1][1].wait()
                issue_scat(i - 1)
            init_h[i].wait()
            gadd_h[i] = (
                pltpu.async_copy(
                    x_hbm.at[idx_a_v.at[i]],
                    bufs[p].at[pl.ds(0, 104)], s_gadd[p], add=True),
                pltpu.async_copy(
                    x_hbm.at[idx_b_v.at[i]],
                    bufs[p].at[pl.ds(104, 96)], s_gadd[p], add=True),
            )
            if i + 1 < ppw:
                if i >= 1:
                    scat_h[i - 1].wait()
                issue_init(i + 1)
        gadd_h[ppw - 1][0].wait()
        gadd_h[ppw - 1][1].wait()
        issue_scat(ppw - 1)
        scat_h[ppw - 2].wait()
        scat_h[ppw - 1].wait()

    return sc_kernel


def kernel(x, pos_table):
    b, l, d = x.shape
    lw = l // WIDE
    dw = d * WIDE
    x2 = x.reshape(b * lw, WIDE, d)
    pos2 = jnp.tile(pos_table.reshape(lw, WIDE, d), (PAIR, 1, 1))
    rows = jnp.arange(b * lw, dtype=jnp.int32).reshape(b // PAIR, PAIR * lw)
    idx_a = rows[:, :104]
    idx_b = rows[:, 104:]
    out = _make_sc_kernel(b, lw, d)(x2, pos2, idx_a, idx_b)
    return out.reshape(b, l, d)
